# baseline (speedup 1.0000x reference)
import functools

import jax
import jax.numpy as jnp
from jax import lax
from jax.experimental import pallas as pl
from jax.experimental.pallas import tpu as pltpu
from jax.experimental.pallas import tpu_sc as plsc

VALUE_MAX = 300.0
NUM_BINS = 601
LANES = 16
NUM_WORKERS = 32
SPLIT = 304


def _two_hot(s):
    pos = jnp.clip(s, -VALUE_MAX, VALUE_MAX) + VALUE_MAX
    li = pos.astype(jnp.int32)
    uw = pos - li.astype(jnp.float32)
    lw = 1.0 - uw
    ui = jnp.minimum(li + 1, NUM_BINS - 1)
    return li, ui, lw, uw


def _make_sc_kernel(batch):
    cols_per_worker = batch // NUM_WORKERS
    chunk_cols = 128
    n_chunks = cols_per_worker // chunk_cols
    groups = chunk_cols // LANES
    halves = ((0, SPLIT), (SPLIT, NUM_BINS - SPLIT))

    mesh = plsc.VectorSubcoreMesh(core_axis_name="c", subcore_axis_name="s")

    @functools.partial(
        pl.kernel,
        out_type=jax.ShapeDtypeStruct((NUM_BINS, batch), jnp.float32),
        mesh=mesh,
        scratch_types=[
            pltpu.VMEM((cols_per_worker,), jnp.float32),
            pltpu.VMEM((halves[0][1], chunk_cols), jnp.float32),
            pltpu.VMEM((halves[1][1], chunk_cols), jnp.float32),
            pltpu.SemaphoreType.DMA,
            pltpu.SemaphoreType.DMA,
        ],
        compiler_params=pltpu.CompilerParams(needs_layout_passes=False),
    )
    def body(scalar_hbm, out_hbm, scal_v, buf_lo, buf_hi, sem_lo, sem_hi):
        wid = lax.axis_index("c") * 16 + lax.axis_index("s")
        col0 = wid * cols_per_worker

        pltpu.sync_copy(scalar_hbm.at[pl.ds(col0, cols_per_worker)], scal_v)

        zeros16 = jnp.zeros((LANES,), jnp.float32)
        lane = lax.iota(jnp.int32, LANES)
        slabs = (
            (buf_lo, sem_lo, halves[0]),
            (buf_hi, sem_hi, halves[1]),
        )

        def zbody(r, carry):
            for buf, _, (_, nrows) in slabs:
                for k in range(groups):

                    @pl.when(r < nrows)
                    def _():
                        buf[r, pl.ds(k * LANES, LANES)] = zeros16

            return carry

        lax.fori_loop(0, halves[0][1], zbody, 0)

        def half_idx(idx, row0, nrows):
            return jnp.clip(idx - row0, 0, nrows - 1)

        def scatter_half(c, buf, row0, nrows, values):
            for g in range(groups):
                s = scal_v[pl.ds(c * chunk_cols + g * LANES, LANES)]
                li, ui, lw, uw = _two_hot(s)
                cols = lane + g * LANES
                in_u = (ui >= row0) & (ui < row0 + nrows)
                in_l = (li >= row0) & (li < row0 + nrows)
                uval = uw if values else zeros16
                lval = lw if values else zeros16
                plsc.store_scatter(
                    buf, [half_idx(ui, row0, nrows), cols], uval, mask=in_u)
                plsc.store_scatter(
                    buf, [half_idx(li, row0, nrows), cols], lval, mask=in_l)

        def dma_dst(c, row0, nrows):
            return out_hbm.at[pl.ds(row0, nrows),
                              pl.ds(col0 + c * chunk_cols, chunk_cols)]

        for buf, sem, (row0, nrows) in slabs:
            scatter_half(0, buf, row0, nrows, values=True)
            pltpu.async_copy(buf, dma_dst(0, row0, nrows), sem)

        def chunk_body(c, carry):
            for buf, sem, (row0, nrows) in slabs:
                pltpu.make_async_copy(
                    buf, dma_dst(c, row0, nrows), sem).wait()
                scatter_half(c - 1, buf, row0, nrows, values=False)
                scatter_half(c, buf, row0, nrows, values=True)
                pltpu.async_copy(buf, dma_dst(c, row0, nrows), sem)
            return carry

        lax.fori_loop(1, n_chunks, chunk_body, 0)

        for buf, sem, (row0, nrows) in slabs:
            pltpu.make_async_copy(
                buf, dma_dst(n_chunks - 1, row0, nrows), sem).wait()

    return body


def kernel(scalar):
    out_t = _make_sc_kernel(scalar.shape[0])(scalar)
    return out_t.T


# speedup vs baseline: 1.0027x; 1.0027x over previous
import functools

import jax
import jax.numpy as jnp
from jax import lax
from jax.experimental import pallas as pl
from jax.experimental.pallas import tpu as pltpu
from jax.experimental.pallas import tpu_sc as plsc

VALUE_MAX = 300.0
NUM_BINS = 601
LANES = 16
NUM_WORKERS = 32


def _two_hot(s):
    pos = jnp.clip(s, -VALUE_MAX, VALUE_MAX) + VALUE_MAX
    li = pos.astype(jnp.int32)
    uw = pos - li.astype(jnp.float32)
    lw = 1.0 - uw
    ui = jnp.minimum(li + 1, NUM_BINS - 1)
    return li, ui, lw, uw


def _make_sc_kernel(batch):
    cols_per_worker = batch // NUM_WORKERS
    chunk_cols = 128
    n_chunks = cols_per_worker // chunk_cols
    groups = chunk_cols // LANES

    mesh = plsc.VectorSubcoreMesh(core_axis_name="c", subcore_axis_name="s")

    @functools.partial(
        pl.kernel,
        out_type=jax.ShapeDtypeStruct((NUM_BINS, batch), jnp.float32),
        mesh=mesh,
        scratch_types=[
            pltpu.VMEM((cols_per_worker,), jnp.float32),
            pltpu.VMEM((NUM_BINS, chunk_cols), jnp.float32),
        ],
        compiler_params=pltpu.CompilerParams(needs_layout_passes=False),
    )
    def body(scalar_hbm, out_hbm, scal_v, buf):
        wid = lax.axis_index("c") * 16 + lax.axis_index("s")
        col0 = wid * cols_per_worker

        pltpu.sync_copy(scalar_hbm.at[pl.ds(col0, cols_per_worker)], scal_v)

        zeros16 = jnp.zeros((LANES,), jnp.float32)
        lane = lax.iota(jnp.int32, LANES)

        def zbody(r, carry):
            for k in range(groups):
                buf[r, pl.ds(k * LANES, LANES)] = zeros16
            return carry

        lax.fori_loop(0, NUM_BINS, zbody, 0)

        def chunk_body(c, carry):
            for g in range(groups):
                s = scal_v[pl.ds(c * chunk_cols + g * LANES, LANES)]
                li, ui, lw, uw = _two_hot(s)
                cols = lane + g * LANES
                plsc.store_scatter(buf, [ui, cols], uw)
                plsc.store_scatter(buf, [li, cols], lw)
            pltpu.sync_copy(
                buf, out_hbm.at[:, pl.ds(col0 + c * chunk_cols, chunk_cols)])
            for g in range(groups):
                s = scal_v[pl.ds(c * chunk_cols + g * LANES, LANES)]
                li, ui, _, _ = _two_hot(s)
                cols = lane + g * LANES
                plsc.store_scatter(buf, [ui, cols], zeros16)
                plsc.store_scatter(buf, [li, cols], zeros16)
            return carry

        lax.fori_loop(0, n_chunks, chunk_body, 0)

    return body


def kernel(scalar):
    out_t = _make_sc_kernel(scalar.shape[0])(scalar)
    return out_t.T
